# baseline (speedup 1.0000x reference)
import functools

import jax
import jax.numpy as jnp
from jax import lax
from jax.experimental import pallas as pl
from jax.experimental.pallas import tpu as pltpu
from jax.experimental.pallas import tpu_sc as plsc

N = 10000
E = 320000
E1 = 81920
E2 = E - E1
D = 128
D_EDGE = 16


def _edge_lin_body(ea_ref, we_ref, be_ref, out_ref):
    out_ref[...] = (
        jnp.dot(ea_ref[...], we_ref[...], preferred_element_type=jnp.float32)
        + be_ref[...]
    )


def _edge_linear(edge_attr, We, be, blk):
    ne = edge_attr.shape[0]
    return pl.pallas_call(
        _edge_lin_body,
        grid=(ne // blk,),
        in_specs=[
            pl.BlockSpec((blk, D_EDGE), lambda i: (i, 0)),
            pl.BlockSpec((D_EDGE, D), lambda i: (0, 0)),
            pl.BlockSpec((1, D), lambda i: (0, 0)),
        ],
        out_specs=pl.BlockSpec((blk, D), lambda i: (i, 0)),
        out_shape=jax.ShapeDtypeStruct((ne, D), jnp.float32),
    )(edge_attr, We, be.reshape(1, D))


_NW = 32
_K = 40
_RB = 16
_RPT = 624


def _sc_aggregate(x, srcw, dstw, e, epw):
    nchunk = epw // _K
    mesh = plsc.VectorSubcoreMesh(core_axis_name="c", subcore_axis_name="s")

    @functools.partial(
        pl.kernel,
        mesh=mesh,
        out_type=jax.ShapeDtypeStruct((2, N, D), jnp.float32),
        scratch_types=[
            pltpu.VMEM((epw,), jnp.int32),
            pltpu.VMEM((epw,), jnp.int32),
            pltpu.VMEM((_K, D), jnp.float32),
            pltpu.VMEM((_K, D), jnp.float32),
            pltpu.VMEM((_K, D), jnp.float32),
            pltpu.VMEM((_K, D), jnp.float32),
            pltpu.VMEM((_K, D), jnp.float32),
            pltpu.VMEM((_K, D), jnp.float32),
            pltpu.VMEM_SHARED((N, D), jnp.float32),
            pltpu.SemaphoreType.DMA,
            pltpu.SemaphoreType.DMA,
            pltpu.SemaphoreType.DMA,
            pltpu.SemaphoreType.DMA,
            pltpu.SemaphoreType.DMA,
        ],
    )
    def k(x_hbm, src_hbm, dst_hbm, e_hbm, out_hbm,
          src_v, dst_v, xj0, xj1, e0, e1, m0, m1, acc_sh,
          gsem0, gsem1, ssem0, ssem1, isem):
        c = lax.axis_index("c")
        s = lax.axis_index("s")
        w = c * 16 + s
        base0 = w * epw
        xj = (xj0, xj1)
        eb = (e0, e1)
        mb = (m0, m1)
        gsem = (gsem0, gsem1)
        ssem = (ssem0, ssem1)

        nper = _RPT // _RB + jnp.where(s == 15, 1, 0)
        zero16 = jnp.zeros((16,), jnp.float32)

        def zrow(i, carry):
            for j in range(8):
                m0[i, pl.ds(j * 16, 16)] = zero16
            return carry

        lax.fori_loop(0, _RB, zrow, 0)

        def zcopy(t, carry):
            pltpu.sync_copy(m0.at[pl.ds(0, _RB)],
                            acc_sh.at[pl.ds(s * _RPT + t * _RB, _RB)])
            return carry

        lax.fori_loop(0, nper, zcopy, 0)

        pltpu.async_copy(src_hbm.at[w], src_v, isem).wait()
        pltpu.async_copy(dst_hbm.at[w], dst_v, isem).wait()
        plsc.subcore_barrier()

        def issue(g, b):
            pltpu.async_copy(x_hbm.at[src_v.at[pl.ds(g * _K, _K)]], xj[b],
                             gsem[b])
            pltpu.async_copy(e_hbm.at[pl.ds(base0 + g * _K, _K)], eb[b],
                             gsem[b])

        def wait_gather(b):
            pltpu.make_async_copy(e_hbm.at[pl.ds(0, _K)], eb[b],
                                  gsem[b]).wait()
            pltpu.make_async_copy(e_hbm.at[pl.ds(0, _K)], xj[b],
                                  gsem[b]).wait()

        def compute(b):
            xb, ebb, mbb = xj[b], eb[b], mb[b]

            def row(i, cc):
                for u in range(4):
                    r = 4 * i + u
                    for j in range(8):
                        sl = pl.ds(j * 16, 16)
                        mbb[r, sl] = jnp.maximum(xb[r, sl] + ebb[r, sl], 0.0)
                return cc

            lax.fori_loop(0, _K // 4, row, 0)

        def scatter(g, b):
            pltpu.async_copy(mb[b], acc_sh.at[dst_v.at[pl.ds(g * _K, _K)]],
                             ssem[b], add=True)

        def wait_scatter(b):
            pltpu.make_async_copy(e_hbm.at[pl.ds(0, _K)], mb[b],
                                  ssem[b]).wait()

        issue(0, 0)
        issue(1, 1)
        for g0 in (0, 1):
            wait_gather(g0)
            compute(g0)
            scatter(g0, g0)
            issue(g0 + 2, g0)

        def body(i, carry):
            t = 2 + 2 * i
            for b in (0, 1):
                g = t + b
                wait_gather(b)
                wait_scatter(b)
                compute(b)
                scatter(g, b)
                issue(g + 2, b)
            return carry

        lax.fori_loop(0, (nchunk - 4) // 2, body, 0)

        for g, b in ((nchunk - 2, 0), (nchunk - 1, 1)):
            wait_gather(b)
            wait_scatter(b)
            compute(b)
            scatter(g, b)
        wait_scatter(0)
        wait_scatter(1)
        plsc.subcore_barrier()

        def drain(t, carry):
            off = s * _RPT + t * _RB
            pltpu.sync_copy(acc_sh.at[pl.ds(off, _RB)], m0.at[pl.ds(0, _RB)])
            pltpu.sync_copy(m0.at[pl.ds(0, _RB)], out_hbm.at[c, pl.ds(off, _RB)])
            return carry

        lax.fori_loop(0, nper, drain, 0)

    return k(x, srcw, dstw, e)


_BN = 2000


def _mlp_body(x_ref, aa_ref, ab_ref, w1_ref, b1_ref, w2_ref, b2_ref, lg_ref,
              lb_ref, out_ref):
    xb = x_ref[...]
    h = xb + (aa_ref[0] + aa_ref[1]) + (ab_ref[0] + ab_ref[1])
    h1 = jnp.maximum(
        jnp.dot(h, w1_ref[...], preferred_element_type=jnp.float32)
        + b1_ref[...], 0.0)
    h2 = (jnp.dot(h1, w2_ref[...], preferred_element_type=jnp.float32)
          + b2_ref[...])
    mu = jnp.mean(h2, axis=-1, keepdims=True)
    var = jnp.mean(jnp.square(h2 - mu), axis=-1, keepdims=True)
    hn = (h2 - mu) * lax.rsqrt(var + 1e-5) * lg_ref[...] + lb_ref[...]
    z = hn + xb
    out_ref[...] = jnp.where(z > 0, z, jnp.exp(jnp.minimum(z, 0.0)) - 1.0)


def _mlp(x, agg_a, agg_b, W1f, b1f, W2, b2, ln_gamma, ln_beta):
    return pl.pallas_call(
        _mlp_body,
        grid=(N // _BN,),
        in_specs=[
            pl.BlockSpec((_BN, D), lambda i: (i, 0)),
            pl.BlockSpec((2, _BN, D), lambda i: (0, i, 0)),
            pl.BlockSpec((2, _BN, D), lambda i: (0, i, 0)),
            pl.BlockSpec((D, D), lambda i: (0, 0)),
            pl.BlockSpec((1, D), lambda i: (0, 0)),
            pl.BlockSpec((D, D), lambda i: (0, 0)),
            pl.BlockSpec((1, D), lambda i: (0, 0)),
            pl.BlockSpec((1, D), lambda i: (0, 0)),
            pl.BlockSpec((1, D), lambda i: (0, 0)),
        ],
        out_specs=pl.BlockSpec((_BN, D), lambda i: (i, 0)),
        out_shape=jax.ShapeDtypeStruct((N, D), jnp.float32),
    )(x, agg_a, agg_b, W1f, b1f.reshape(1, D), W2, b2.reshape(1, D),
      ln_gamma.reshape(1, D), ln_beta.reshape(1, D))


def kernel(x, edge_index, edge_attr, We, be, W1, b1, bn_gamma, bn_beta,
           bn_mean, bn_var, W2, b2, ln_gamma, ln_beta):
    epw1 = E1 // _NW
    epw2 = E2 // _NW
    src_a = edge_index[0, :E1].reshape(_NW, epw1)
    dst_a = edge_index[1, :E1].reshape(_NW, epw1)
    src_b = edge_index[0, E1:].reshape(_NW, epw2)
    dst_b = edge_index[1, E1:].reshape(_NW, epw2)
    e_a = _edge_linear(edge_attr[:E1], We, be, 8192)
    e_b = _edge_linear(edge_attr[E1:], We, be, 7680)
    agg_a = _sc_aggregate(x, src_a, dst_a, e_a, epw1)
    agg_b = _sc_aggregate(x, src_b, dst_b, e_b, epw2)
    scale = bn_gamma * lax.rsqrt(bn_var + 1e-5)
    W1f = W1 * scale[None, :]
    b1f = (b1 - bn_mean) * scale + bn_beta
    return _mlp(x, agg_a, agg_b, W1f, b1f, W2, b2, ln_gamma, ln_beta)


# speedup vs baseline: 1.0514x; 1.0514x over previous
import functools

import jax
import jax.numpy as jnp
from jax import lax
from jax.experimental import pallas as pl
from jax.experimental.pallas import tpu as pltpu
from jax.experimental.pallas import tpu_sc as plsc

N = 10000
E = 320000
E1 = 161280
E2 = E - E1
D = 128
D_EDGE = 16


def _edge_lin_body(ea_ref, we_ref, be_ref, out_ref):
    out_ref[...] = (
        jnp.dot(ea_ref[...], we_ref[...], preferred_element_type=jnp.float32)
        + be_ref[...]
    )


def _edge_linear(edge_attr, We, be, blk):
    ne = edge_attr.shape[0]
    return pl.pallas_call(
        _edge_lin_body,
        grid=(ne // blk,),
        in_specs=[
            pl.BlockSpec((blk, D_EDGE), lambda i: (i, 0)),
            pl.BlockSpec((D_EDGE, D), lambda i: (0, 0)),
            pl.BlockSpec((1, D), lambda i: (0, 0)),
        ],
        out_specs=pl.BlockSpec((blk, D), lambda i: (i, 0)),
        out_shape=jax.ShapeDtypeStruct((ne, D), jnp.float32),
    )(edge_attr, We, be.reshape(1, D))


_NW = 32
_K = 40
_RB = 16
_RPT = 624


def _sc_aggregate(x, srcw, dstw, e, epw):
    nchunk = epw // _K
    mesh = plsc.VectorSubcoreMesh(core_axis_name="c", subcore_axis_name="s")

    @functools.partial(
        pl.kernel,
        mesh=mesh,
        out_type=jax.ShapeDtypeStruct((2, N, D), jnp.float32),
        scratch_types=[
            pltpu.VMEM((epw,), jnp.int32),
            pltpu.VMEM((epw,), jnp.int32),
            pltpu.VMEM((_K, D), jnp.float32),
            pltpu.VMEM((_K, D), jnp.float32),
            pltpu.VMEM((_K, D), jnp.float32),
            pltpu.VMEM((_K, D), jnp.float32),
            pltpu.VMEM((_K, D), jnp.float32),
            pltpu.VMEM((_K, D), jnp.float32),
            pltpu.VMEM_SHARED((N, D), jnp.float32),
            pltpu.SemaphoreType.DMA,
            pltpu.SemaphoreType.DMA,
            pltpu.SemaphoreType.DMA,
            pltpu.SemaphoreType.DMA,
            pltpu.SemaphoreType.DMA,
        ],
    )
    def k(x_hbm, src_hbm, dst_hbm, e_hbm, out_hbm,
          src_v, dst_v, xj0, xj1, e0, e1, m0, m1, acc_sh,
          gsem0, gsem1, ssem0, ssem1, isem):
        c = lax.axis_index("c")
        s = lax.axis_index("s")
        w = c * 16 + s
        base0 = w * epw
        xj = (xj0, xj1)
        eb = (e0, e1)
        mb = (m0, m1)
        gsem = (gsem0, gsem1)
        ssem = (ssem0, ssem1)

        nper = _RPT // _RB + jnp.where(s == 15, 1, 0)
        zero16 = jnp.zeros((16,), jnp.float32)

        def zrow(i, carry):
            for j in range(8):
                m0[i, pl.ds(j * 16, 16)] = zero16
            return carry

        lax.fori_loop(0, _RB, zrow, 0)

        def zcopy(t, carry):
            pltpu.sync_copy(m0.at[pl.ds(0, _RB)],
                            acc_sh.at[pl.ds(s * _RPT + t * _RB, _RB)])
            return carry

        lax.fori_loop(0, nper, zcopy, 0)

        pltpu.async_copy(src_hbm.at[w], src_v, isem).wait()
        pltpu.async_copy(dst_hbm.at[w], dst_v, isem).wait()
        plsc.subcore_barrier()

        def issue(g, b):
            pltpu.async_copy(x_hbm.at[src_v.at[pl.ds(g * _K, _K)]], xj[b],
                             gsem[b])
            pltpu.async_copy(e_hbm.at[pl.ds(base0 + g * _K, _K)], eb[b],
                             gsem[b])

        def wait_gather(b):
            pltpu.make_async_copy(e_hbm.at[pl.ds(0, _K)], eb[b],
                                  gsem[b]).wait()
            pltpu.make_async_copy(e_hbm.at[pl.ds(0, _K)], xj[b],
                                  gsem[b]).wait()

        def compute(b):
            xb, ebb, mbb = xj[b], eb[b], mb[b]

            def row(i, cc):
                for u in range(4):
                    r = 4 * i + u
                    for j in range(8):
                        sl = pl.ds(j * 16, 16)
                        mbb[r, sl] = jnp.maximum(xb[r, sl] + ebb[r, sl], 0.0)
                return cc

            lax.fori_loop(0, _K // 4, row, 0)

        def scatter(g, b):
            pltpu.async_copy(mb[b], acc_sh.at[dst_v.at[pl.ds(g * _K, _K)]],
                             ssem[b], add=True)

        def wait_scatter(b):
            pltpu.make_async_copy(e_hbm.at[pl.ds(0, _K)], mb[b],
                                  ssem[b]).wait()

        issue(0, 0)
        issue(1, 1)
        for g0 in (0, 1):
            wait_gather(g0)
            compute(g0)
            scatter(g0, g0)
            issue(g0 + 2, g0)

        def body(i, carry):
            t = 2 + 2 * i
            for b in (0, 1):
                g = t + b
                wait_gather(b)
                wait_scatter(b)
                compute(b)
                scatter(g, b)
                issue(g + 2, b)
            return carry

        lax.fori_loop(0, (nchunk - 4) // 2, body, 0)

        for g, b in ((nchunk - 2, 0), (nchunk - 1, 1)):
            wait_gather(b)
            wait_scatter(b)
            compute(b)
            scatter(g, b)
        wait_scatter(0)
        wait_scatter(1)
        plsc.subcore_barrier()

        def drain(t, carry):
            off = s * _RPT + t * _RB
            pltpu.sync_copy(acc_sh.at[pl.ds(off, _RB)], m0.at[pl.ds(0, _RB)])
            pltpu.sync_copy(m0.at[pl.ds(0, _RB)], out_hbm.at[c, pl.ds(off, _RB)])
            return carry

        lax.fori_loop(0, nper, drain, 0)

    return k(x, srcw, dstw, e)


_BN = 2000


def _mlp_body(x_ref, aa_ref, ab_ref, w1_ref, b1_ref, w2_ref, b2_ref, lg_ref,
              lb_ref, out_ref):
    xb = x_ref[...]
    h = xb + (aa_ref[0] + aa_ref[1]) + (ab_ref[0] + ab_ref[1])
    h1 = jnp.maximum(
        jnp.dot(h, w1_ref[...], preferred_element_type=jnp.float32)
        + b1_ref[...], 0.0)
    h2 = (jnp.dot(h1, w2_ref[...], preferred_element_type=jnp.float32)
          + b2_ref[...])
    mu = jnp.mean(h2, axis=-1, keepdims=True)
    var = jnp.mean(jnp.square(h2 - mu), axis=-1, keepdims=True)
    hn = (h2 - mu) * lax.rsqrt(var + 1e-5) * lg_ref[...] + lb_ref[...]
    z = hn + xb
    out_ref[...] = jnp.where(z > 0, z, jnp.exp(jnp.minimum(z, 0.0)) - 1.0)


def _mlp(x, agg_a, agg_b, W1f, b1f, W2, b2, ln_gamma, ln_beta):
    return pl.pallas_call(
        _mlp_body,
        grid=(N // _BN,),
        in_specs=[
            pl.BlockSpec((_BN, D), lambda i: (i, 0)),
            pl.BlockSpec((2, _BN, D), lambda i: (0, i, 0)),
            pl.BlockSpec((2, _BN, D), lambda i: (0, i, 0)),
            pl.BlockSpec((D, D), lambda i: (0, 0)),
            pl.BlockSpec((1, D), lambda i: (0, 0)),
            pl.BlockSpec((D, D), lambda i: (0, 0)),
            pl.BlockSpec((1, D), lambda i: (0, 0)),
            pl.BlockSpec((1, D), lambda i: (0, 0)),
            pl.BlockSpec((1, D), lambda i: (0, 0)),
        ],
        out_specs=pl.BlockSpec((_BN, D), lambda i: (i, 0)),
        out_shape=jax.ShapeDtypeStruct((N, D), jnp.float32),
    )(x, agg_a, agg_b, W1f, b1f.reshape(1, D), W2, b2.reshape(1, D),
      ln_gamma.reshape(1, D), ln_beta.reshape(1, D))


def kernel(x, edge_index, edge_attr, We, be, W1, b1, bn_gamma, bn_beta,
           bn_mean, bn_var, W2, b2, ln_gamma, ln_beta):
    epw1 = E1 // _NW
    epw2 = E2 // _NW
    src_a = edge_index[0, :E1].reshape(_NW, epw1)
    dst_a = edge_index[1, :E1].reshape(_NW, epw1)
    src_b = edge_index[0, E1:].reshape(_NW, epw2)
    dst_b = edge_index[1, E1:].reshape(_NW, epw2)
    e_a = _edge_linear(edge_attr[:E1], We, be, 8064)
    e_b = _edge_linear(edge_attr[E1:], We, be, 7936)
    agg_a = _sc_aggregate(x, src_a, dst_a, e_a, epw1)
    agg_b = _sc_aggregate(x, src_b, dst_b, e_b, epw2)
    scale = bn_gamma * lax.rsqrt(bn_var + 1e-5)
    W1f = W1 * scale[None, :]
    b1f = (b1 - bn_mean) * scale + bn_beta
    return _mlp(x, agg_a, agg_b, W1f, b1f, W2, b2, ln_gamma, ln_beta)
